# baseline (speedup 1.0000x reference)
import functools

import jax
import jax.numpy as jnp
from jax import lax
from jax.experimental import pallas as pl
from jax.experimental.pallas import tpu as pltpu
from jax.experimental.pallas import tpu_sc as plsc

_B = 32
_C = 3
_H = 224
_W = 224
_PC = 14
_PP = 16
_D = 192
_P = _PC * _PC
_K = _C * _PP * _PP
_PPAD = 208
_HW = _H * _W


def _erf(z):
    s = jnp.sign(z)
    a = jnp.abs(z)
    t = 1.0 / (1.0 + 0.3275911 * a)
    poly = t * (0.254829592 + t * (-0.284496736 + t * (1.421413741
                + t * (-1.453152027 + t * 1.061405429))))
    return s * (1.0 - poly * jnp.exp(-a * a))


def _layernorm(x, w, b, eps=1e-5):
    m = jnp.mean(x, axis=-1, keepdims=True)
    xc = x - m
    v = jnp.mean(xc * xc, axis=-1, keepdims=True)
    return xc * jax.lax.rsqrt(v + eps) * w + b


def _head_body(patches_ref, cw_ref, cb_ref, pw_ref, pb_ref, offw_ref,
               idx_ref, wgt_ref):
    p = patches_ref[0]
    feat = jnp.dot(p, cw_ref[...], preferred_element_type=jnp.float32)
    feat = feat + cb_ref[...]
    feat = _layernorm(feat, pw_ref[...], pb_ref[...])
    g = 0.5 * feat * (1.0 + _erf(feat * 0.7071067811865476))
    off = lax.dot_general(g, offw_ref[...], (((1,), (1,)), ((), ())),
                          preferred_element_type=jnp.float32)

    pid = lax.broadcasted_iota(jnp.int32, (_P, 1), 0)
    gx = ((pid % _PC).astype(jnp.float32) + 0.5) / _PC
    gy = ((pid // _PC).astype(jnp.float32) + 0.5) / _PC
    w0 = _PP / float(_W)
    h0 = _PP / float(_H)
    cx = gx + off[:, 0:1] * w0
    cy = gy + off[:, 1:2] * h0
    px = cx * _W - 0.5
    py = cy * _H - 0.5
    x0f = jnp.floor(px)
    y0f = jnp.floor(py)
    wx1 = px - x0f
    wy1 = py - y0f
    x0 = x0f.astype(jnp.int32)
    y0 = y0f.astype(jnp.int32)

    idx_cols = []
    wgt_cols = []
    for dy, dx, wfac in ((0, 0, (1.0 - wx1) * (1.0 - wy1)),
                         (0, 1, wx1 * (1.0 - wy1)),
                         (1, 0, (1.0 - wx1) * wy1),
                         (1, 1, wx1 * wy1)):
        ix = x0 + dx
        iy = y0 + dy
        valid = ((ix >= 0) & (ix < _W) & (iy >= 0) & (iy < _H))
        flat = (jnp.clip(iy, 0, _H - 1) * _W + jnp.clip(ix, 0, _W - 1))
        idx_cols.append(flat)
        wgt_cols.append(wfac * valid.astype(jnp.float32))
    idx_ref[0] = jnp.concatenate(idx_cols, axis=1)
    wgt_ref[0] = jnp.concatenate(wgt_cols, axis=1)


def _run_head(patches, cwf, cb, pw, pb, offw):
    return pl.pallas_call(
        _head_body,
        grid=(_B,),
        in_specs=[
            pl.BlockSpec((1, _P, _K), lambda b: (b, 0, 0)),
            pl.BlockSpec((_K, _D), lambda b: (0, 0)),
            pl.BlockSpec((1, _D), lambda b: (0, 0)),
            pl.BlockSpec((1, _D), lambda b: (0, 0)),
            pl.BlockSpec((1, _D), lambda b: (0, 0)),
            pl.BlockSpec((4, _D), lambda b: (0, 0)),
        ],
        out_specs=[
            pl.BlockSpec((1, _P, 4), lambda b: (b, 0, 0)),
            pl.BlockSpec((1, _P, 4), lambda b: (b, 0, 0)),
        ],
        out_shape=[
            jax.ShapeDtypeStruct((_B, _P, 4), jnp.int32),
            jax.ShapeDtypeStruct((_B, _P, 4), jnp.float32),
        ],
    )(patches, cwf, cb, pw, pb, offw)


def _sc_bilinear(xplanes, idx, wgt):
    mesh = plsc.VectorSubcoreMesh(core_axis_name="c", subcore_axis_name="s")

    @functools.partial(
        pl.kernel,
        mesh=mesh,
        compiler_params=pltpu.CompilerParams(needs_layout_passes=False,
                                             use_tc_tiling_on_sc=True),
        out_type=jax.ShapeDtypeStruct((_B, _C, _PPAD), jnp.float32),
        scratch_types=[
            pltpu.VMEM((_HW,), jnp.float32),
            pltpu.VMEM((4, _PPAD), jnp.int32),
            pltpu.VMEM((4, _PPAD), jnp.float32),
            pltpu.VMEM((_C, _PPAD), jnp.float32),
        ],
    )
    def k(x_hbm, idx_hbm, wgt_hbm, out_hbm, plane_v, idx_v, wgt_v, samp_v):
        b = lax.axis_index("s") * 2 + lax.axis_index("c")
        pltpu.sync_copy(idx_hbm.at[b], idx_v)
        pltpu.sync_copy(wgt_hbm.at[b], wgt_v)
        for c in range(_C):
            pltpu.sync_copy(x_hbm.at[b * _C + c], plane_v)
            for j in range(_PPAD // 16):
                sl = pl.ds(j * 16, 16)
                acc = plsc.load_gather(plane_v, [idx_v[0, sl]]) * wgt_v[0, sl]
                for t in range(1, 4):
                    acc = acc + (plsc.load_gather(plane_v, [idx_v[t, sl]])
                                 * wgt_v[t, sl])
                samp_v[c, sl] = acc
        pltpu.sync_copy(samp_v, out_hbm.at[b])

    return k(xplanes, idx, wgt)


def _tail_body(samp_ref, ow_ref, ob_ref, nw_ref, nb_ref, out_ref):
    s = samp_ref[0]
    ow = ow_ref[...]
    kk = lax.broadcasted_iota(jnp.int32, (_K, _C), 0) % _C
    cc = lax.broadcasted_iota(jnp.int32, (_K, _C), 1)
    m = (kk == cc).astype(jnp.float32)
    a = jnp.dot(ow, m, preferred_element_type=jnp.float32)
    pre = lax.dot_general(s, a, (((1,), (1,)), ((), ())),
                          preferred_element_type=jnp.float32)
    pre = pre + ob_ref[...]
    out_ref[0] = _layernorm(pre, nw_ref[...], nb_ref[...])


def _run_tail(samp3, ow, ob, nw, nb):
    return pl.pallas_call(
        _tail_body,
        grid=(_B,),
        in_specs=[
            pl.BlockSpec((1, _P, _C), lambda b: (b, 0, 0)),
            pl.BlockSpec((_D, _K), lambda b: (0, 0)),
            pl.BlockSpec((1, _D), lambda b: (0, 0)),
            pl.BlockSpec((1, _D), lambda b: (0, 0)),
            pl.BlockSpec((1, _D), lambda b: (0, 0)),
        ],
        out_specs=pl.BlockSpec((1, _P, _D), lambda b: (b, 0, 0)),
        out_shape=jax.ShapeDtypeStruct((_B, _P, _D), jnp.float32),
    )(samp3, ow, ob, nw, nb)


def kernel(x, conv_w, conv_b, pe_norm_w, pe_norm_b, off_w, out_w, out_b,
           norm_w, norm_b):
    patches = (x.astype(jnp.bfloat16)
               .reshape(_B, _C, _PC, _PP, _PC, _PP)
               .transpose(0, 2, 4, 1, 3, 5)
               .reshape(_B, _P, _K))
    cwf = conv_w.reshape(_D, _K).T.astype(jnp.bfloat16)
    idx, wgt = _run_head(patches, cwf,
                         conv_b.reshape(1, _D),
                         pe_norm_w.reshape(1, _D),
                         pe_norm_b.reshape(1, _D),
                         off_w)
    pad = _PPAD - _P
    idx_t = jnp.pad(idx.transpose(0, 2, 1), ((0, 0), (0, 0), (0, pad)))
    wgt_t = jnp.pad(wgt.transpose(0, 2, 1), ((0, 0), (0, 0), (0, pad)))
    samp = _sc_bilinear(x.reshape(_B * _C, _HW), idx_t, wgt_t)
    samp3 = samp[:, :, :_P].transpose(0, 2, 1)
    return _run_tail(samp3, out_w,
                     out_b.reshape(1, _D),
                     norm_w.reshape(1, _D),
                     norm_b.reshape(1, _D))


# speedup vs baseline: 1.1198x; 1.1198x over previous
import functools

import jax
import jax.numpy as jnp
from jax import lax
from jax.experimental import pallas as pl
from jax.experimental.pallas import tpu as pltpu
from jax.experimental.pallas import tpu_sc as plsc

_B = 32
_C = 3
_H = 224
_W = 224
_PC = 14
_PP = 16
_D = 192
_P = _PC * _PC
_K = _C * _PP * _PP
_PPAD = 208
_HW = _H * _W
_GB = 8
_GN = _B // _GB


def _erf(z):
    s = jnp.sign(z)
    a = jnp.abs(z)
    t = 1.0 / (1.0 + 0.3275911 * a)
    poly = t * (0.254829592 + t * (-0.284496736 + t * (1.421413741
                + t * (-1.453152027 + t * 1.061405429))))
    return s * (1.0 - poly * jnp.exp(-a * a))


def _layernorm(x, w, b, eps=1e-5):
    m = jnp.mean(x, axis=-1, keepdims=True)
    xc = x - m
    v = jnp.mean(xc * xc, axis=-1, keepdims=True)
    return xc * jax.lax.rsqrt(v + eps) * w + b


def _head_body(patches_ref, cw_ref, cb_ref, pw_ref, pb_ref, offw_ref,
               idx_ref, wgt_ref):
    p = patches_ref[...].reshape(_GB * _P, _K)
    feat = jnp.dot(p, cw_ref[...], preferred_element_type=jnp.float32)
    feat = feat + cb_ref[...]
    feat = _layernorm(feat, pw_ref[...], pb_ref[...])
    g = 0.5 * feat * (1.0 + _erf(feat * 0.7071067811865476))
    off = lax.dot_general(g, offw_ref[...], (((1,), (1,)), ((), ())),
                          preferred_element_type=jnp.float32)

    rid = lax.broadcasted_iota(jnp.int32, (_GB * _P, 1), 0)
    pid = rid % _P
    gx = ((pid % _PC).astype(jnp.float32) + 0.5) / _PC
    gy = ((pid // _PC).astype(jnp.float32) + 0.5) / _PC
    w0 = _PP / float(_W)
    h0 = _PP / float(_H)
    cx = gx + off[:, 0:1] * w0
    cy = gy + off[:, 1:2] * h0
    px = cx * _W - 0.5
    py = cy * _H - 0.5
    x0f = jnp.floor(px)
    y0f = jnp.floor(py)
    wx1 = px - x0f
    wy1 = py - y0f
    x0 = x0f.astype(jnp.int32)
    y0 = y0f.astype(jnp.int32)

    idx_cols = []
    wgt_cols = []
    for dy, dx, wfac in ((0, 0, (1.0 - wx1) * (1.0 - wy1)),
                         (0, 1, wx1 * (1.0 - wy1)),
                         (1, 0, (1.0 - wx1) * wy1),
                         (1, 1, wx1 * wy1)):
        ix = x0 + dx
        iy = y0 + dy
        valid = ((ix >= 0) & (ix < _W) & (iy >= 0) & (iy < _H))
        flat = (jnp.clip(iy, 0, _H - 1) * _W + jnp.clip(ix, 0, _W - 1))
        idx_cols.append(flat)
        wgt_cols.append(wfac * valid.astype(jnp.float32))
    idx_ref[...] = jnp.concatenate(idx_cols, axis=1).reshape(_GB, _P, 4)
    wgt_ref[...] = jnp.concatenate(wgt_cols, axis=1).reshape(_GB, _P, 4)


def _run_head(patches, cwf, cb, pw, pb, offw):
    return pl.pallas_call(
        _head_body,
        grid=(_GN,),
        in_specs=[
            pl.BlockSpec((_GB, _P, _K), lambda b: (b, 0, 0)),
            pl.BlockSpec((_K, _D), lambda b: (0, 0)),
            pl.BlockSpec((1, _D), lambda b: (0, 0)),
            pl.BlockSpec((1, _D), lambda b: (0, 0)),
            pl.BlockSpec((1, _D), lambda b: (0, 0)),
            pl.BlockSpec((4, _D), lambda b: (0, 0)),
        ],
        out_specs=[
            pl.BlockSpec((_GB, _P, 4), lambda b: (b, 0, 0)),
            pl.BlockSpec((_GB, _P, 4), lambda b: (b, 0, 0)),
        ],
        out_shape=[
            jax.ShapeDtypeStruct((_B, _P, 4), jnp.int32),
            jax.ShapeDtypeStruct((_B, _P, 4), jnp.float32),
        ],
    )(patches, cwf, cb, pw, pb, offw)


def _sc_bilinear(xplanes, idx, wgt):
    mesh = plsc.VectorSubcoreMesh(core_axis_name="c", subcore_axis_name="s")

    @functools.partial(
        pl.kernel,
        mesh=mesh,
        compiler_params=pltpu.CompilerParams(needs_layout_passes=False),
        out_type=jax.ShapeDtypeStruct((_B, _C, _PPAD), jnp.float32),
        scratch_types=[
            pltpu.VMEM((_HW,), jnp.float32),
            pltpu.VMEM((4, _PPAD), jnp.int32),
            pltpu.VMEM((4, _PPAD), jnp.float32),
            pltpu.VMEM((_C, _PPAD), jnp.float32),
        ],
    )
    def k(x_hbm, idx_hbm, wgt_hbm, out_hbm, plane_v, idx_v, wgt_v, samp_v):
        b = lax.axis_index("s") * 2 + lax.axis_index("c")
        pltpu.sync_copy(idx_hbm.at[b], idx_v)
        pltpu.sync_copy(wgt_hbm.at[b], wgt_v)
        for c in range(_C):
            pltpu.sync_copy(x_hbm.at[b * _C + c], plane_v)
            for j in range(_PPAD // 16):
                sl = pl.ds(j * 16, 16)
                acc = plsc.load_gather(plane_v, [idx_v[0, sl]]) * wgt_v[0, sl]
                for t in range(1, 4):
                    acc = acc + (plsc.load_gather(plane_v, [idx_v[t, sl]])
                                 * wgt_v[t, sl])
                samp_v[c, sl] = acc
        pltpu.sync_copy(samp_v, out_hbm.at[b])

    return k(xplanes, idx, wgt)


def _w3_body(ow_ref, w3_ref):
    ow = ow_ref[...]
    kk = lax.broadcasted_iota(jnp.int32, (_K, _C), 0) % _C
    cc = lax.broadcasted_iota(jnp.int32, (_K, _C), 1)
    m = (kk == cc).astype(jnp.float32)
    w3_ref[...] = jnp.dot(ow, m, preferred_element_type=jnp.float32)


def _run_w3(ow):
    return pl.pallas_call(
        _w3_body,
        out_shape=jax.ShapeDtypeStruct((_D, _C), jnp.float32),
    )(ow)


def _tail_body(samp_ref, w3_ref, ob_ref, nw_ref, nb_ref, out_ref):
    s = samp_ref[...].reshape(_GB * _P, _C)
    pre = lax.dot_general(s, w3_ref[...], (((1,), (1,)), ((), ())),
                          preferred_element_type=jnp.float32)
    pre = pre + ob_ref[...]
    out = _layernorm(pre, nw_ref[...], nb_ref[...])
    out_ref[...] = out.reshape(_GB, _P, _D)


def _run_tail(samp3, w3, ob, nw, nb):
    return pl.pallas_call(
        _tail_body,
        grid=(_GN,),
        in_specs=[
            pl.BlockSpec((_GB, _P, _C), lambda b: (b, 0, 0)),
            pl.BlockSpec((_D, _C), lambda b: (0, 0)),
            pl.BlockSpec((1, _D), lambda b: (0, 0)),
            pl.BlockSpec((1, _D), lambda b: (0, 0)),
            pl.BlockSpec((1, _D), lambda b: (0, 0)),
        ],
        out_specs=pl.BlockSpec((_GB, _P, _D), lambda b: (b, 0, 0)),
        out_shape=jax.ShapeDtypeStruct((_B, _P, _D), jnp.float32),
    )(samp3, w3, ob, nw, nb)


def kernel(x, conv_w, conv_b, pe_norm_w, pe_norm_b, off_w, out_w, out_b,
           norm_w, norm_b):
    z = (x.astype(jnp.bfloat16)
         .reshape(_B, _C, _PC, _PP, _W)
         .transpose(0, 2, 3, 1, 4))
    patches = (z.reshape(_B, _PC, _PP, _C, _PC, _PP)
               .transpose(0, 1, 4, 3, 2, 5)
               .reshape(_B, _P, _K))
    cwf = conv_w.reshape(_D, _K).T.astype(jnp.bfloat16)
    idx, wgt = _run_head(patches, cwf,
                         conv_b.reshape(1, _D),
                         pe_norm_w.reshape(1, _D),
                         pe_norm_b.reshape(1, _D),
                         off_w)
    pad = _PPAD - _P
    idx_t = jnp.pad(idx.transpose(0, 2, 1), ((0, 0), (0, 0), (0, pad)))
    wgt_t = jnp.pad(wgt.transpose(0, 2, 1), ((0, 0), (0, 0), (0, pad)))
    samp = _sc_bilinear(x.reshape(_B * _C, _HW), idx_t, wgt_t)
    samp3 = samp[:, :, :_P].transpose(0, 2, 1)
    w3 = _run_w3(out_w)
    return _run_tail(samp3, w3,
                     out_b.reshape(1, _D),
                     norm_w.reshape(1, _D),
                     norm_b.reshape(1, _D))


# speedup vs baseline: 1.2339x; 1.1019x over previous
import functools

import jax
import jax.numpy as jnp
from jax import lax
from jax.experimental import pallas as pl
from jax.experimental.pallas import tpu as pltpu
from jax.experimental.pallas import tpu_sc as plsc

_B = 32
_C = 3
_H = 224
_W = 224
_PC = 14
_PP = 16
_D = 192
_P = _PC * _PC
_K = _C * _PP * _PP
_PPAD = 208
_HW = _H * _W
_GB = 8
_GN = _B // _GB


def _erf(z):
    s = jnp.sign(z)
    a = jnp.abs(z)
    t = 1.0 / (1.0 + 0.3275911 * a)
    poly = t * (0.254829592 + t * (-0.284496736 + t * (1.421413741
                + t * (-1.453152027 + t * 1.061405429))))
    return s * (1.0 - poly * jnp.exp(-a * a))


def _layernorm(x, w, b, eps=1e-5):
    m = jnp.mean(x, axis=-1, keepdims=True)
    xc = x - m
    v = jnp.mean(xc * xc, axis=-1, keepdims=True)
    return xc * jax.lax.rsqrt(v + eps) * w + b


def _head_body(patches_ref, cw_ref, cb_ref, pw_ref, pb_ref, offw_ref,
               idx_ref, wgt_ref):
    p = patches_ref[...].reshape(_GB * _P, _K)
    feat = jnp.dot(p, cw_ref[...], preferred_element_type=jnp.float32)
    feat = feat + cb_ref[...]
    feat = _layernorm(feat, pw_ref[...], pb_ref[...])
    g = 0.5 * feat * (1.0 + _erf(feat * 0.7071067811865476))
    off = lax.dot_general(g, offw_ref[...], (((1,), (1,)), ((), ())),
                          preferred_element_type=jnp.float32)

    rid = lax.broadcasted_iota(jnp.int32, (_GB * _P, 1), 0)
    pid = rid % _P
    gx = ((pid % _PC).astype(jnp.float32) + 0.5) / _PC
    gy = ((pid // _PC).astype(jnp.float32) + 0.5) / _PC
    w0 = _PP / float(_W)
    h0 = _PP / float(_H)
    cx = gx + off[:, 0:1] * w0
    cy = gy + off[:, 1:2] * h0
    px = cx * _W - 0.5
    py = cy * _H - 0.5
    x0f = jnp.floor(px)
    y0f = jnp.floor(py)
    wx1 = px - x0f
    wy1 = py - y0f
    x0 = x0f.astype(jnp.int32)
    y0 = y0f.astype(jnp.int32)

    idx_cols = []
    wgt_cols = []
    for dy, dx, wfac in ((0, 0, (1.0 - wx1) * (1.0 - wy1)),
                         (0, 1, wx1 * (1.0 - wy1)),
                         (1, 0, (1.0 - wx1) * wy1),
                         (1, 1, wx1 * wy1)):
        ix = x0 + dx
        iy = y0 + dy
        valid = ((ix >= 0) & (ix < _W) & (iy >= 0) & (iy < _H))
        flat = (jnp.clip(iy, 0, _H - 1) * _W + jnp.clip(ix, 0, _W - 1))
        idx_cols.append(flat)
        wgt_cols.append(wfac * valid.astype(jnp.float32))
    idx_ref[...] = jnp.concatenate(idx_cols, axis=1).reshape(_GB, _P, 4)
    wgt_ref[...] = jnp.concatenate(wgt_cols, axis=1).reshape(_GB, _P, 4)


def _run_head(patches, cwf, cb, pw, pb, offw):
    return pl.pallas_call(
        _head_body,
        grid=(_GN,),
        in_specs=[
            pl.BlockSpec((_GB, _P, _K), lambda b: (b, 0, 0)),
            pl.BlockSpec((_K, _D), lambda b: (0, 0)),
            pl.BlockSpec((1, _D), lambda b: (0, 0)),
            pl.BlockSpec((1, _D), lambda b: (0, 0)),
            pl.BlockSpec((1, _D), lambda b: (0, 0)),
            pl.BlockSpec((4, _D), lambda b: (0, 0)),
        ],
        out_specs=[
            pl.BlockSpec((_GB, _P, 4), lambda b: (b, 0, 0)),
            pl.BlockSpec((_GB, _P, 4), lambda b: (b, 0, 0)),
        ],
        out_shape=[
            jax.ShapeDtypeStruct((_B, _P, 4), jnp.int32),
            jax.ShapeDtypeStruct((_B, _P, 4), jnp.float32),
        ],
    )(patches, cwf, cb, pw, pb, offw)


def _sc_bilinear(xplanes, idx, wgt):
    mesh = plsc.VectorSubcoreMesh(core_axis_name="c", subcore_axis_name="s")

    @functools.partial(
        pl.kernel,
        mesh=mesh,
        compiler_params=pltpu.CompilerParams(needs_layout_passes=False),
        out_type=jax.ShapeDtypeStruct((_B, _C, _PPAD), jnp.float32),
        scratch_types=[
            pltpu.VMEM((_HW,), jnp.float32),
            pltpu.VMEM((4, _PPAD), jnp.int32),
            pltpu.VMEM((4, _PPAD), jnp.float32),
            pltpu.VMEM((_C, _PPAD), jnp.float32),
        ],
    )
    def k(x_hbm, idx_hbm, wgt_hbm, out_hbm, plane_v, idx_v, wgt_v, samp_v):
        b = lax.axis_index("s") * 2 + lax.axis_index("c")
        pltpu.sync_copy(idx_hbm.at[b], idx_v)
        pltpu.sync_copy(wgt_hbm.at[b], wgt_v)
        for c in range(_C):
            pltpu.sync_copy(x_hbm.at[b * _C + c], plane_v)
            for j in range(_PPAD // 16):
                sl = pl.ds(j * 16, 16)
                acc = plsc.load_gather(plane_v, [idx_v[0, sl]]) * wgt_v[0, sl]
                for t in range(1, 4):
                    acc = acc + (plsc.load_gather(plane_v, [idx_v[t, sl]])
                                 * wgt_v[t, sl])
                samp_v[c, sl] = acc
        pltpu.sync_copy(samp_v, out_hbm.at[b])

    return k(xplanes, idx, wgt)


def _w3_body(ow_ref, w3_ref):
    ow = ow_ref[...]
    kk = lax.broadcasted_iota(jnp.int32, (_K, _C), 0) % _C
    cc = lax.broadcasted_iota(jnp.int32, (_K, _C), 1)
    m = (kk == cc).astype(jnp.float32)
    w3_ref[...] = jnp.dot(ow, m, preferred_element_type=jnp.float32)


def _run_w3(ow):
    return pl.pallas_call(
        _w3_body,
        out_shape=jax.ShapeDtypeStruct((_D, _C), jnp.float32),
    )(ow)


def _tail_body(samp_ref, w3_ref, ob_ref, nw_ref, nb_ref, out_ref):
    w3 = w3_ref[...]
    obt = ob_ref[...]
    nwt = nw_ref[...]
    nbt = nb_ref[...]
    eps = 1e-5
    for i in range(_GB):
        s = samp_ref[i, :, :_P]
        pre = jnp.dot(w3, s, preferred_element_type=jnp.float32) + obt
        m = jnp.mean(pre, axis=0, keepdims=True)
        xc = pre - m
        v = jnp.mean(xc * xc, axis=0, keepdims=True)
        out_ref[i] = xc * jax.lax.rsqrt(v + eps) * nwt + nbt


def _run_tail(samp, w3, obt, nwt, nbt):
    return pl.pallas_call(
        _tail_body,
        grid=(_GN,),
        in_specs=[
            pl.BlockSpec((_GB, _C, _PPAD), lambda b: (b, 0, 0)),
            pl.BlockSpec((_D, _C), lambda b: (0, 0)),
            pl.BlockSpec((_D, 1), lambda b: (0, 0)),
            pl.BlockSpec((_D, 1), lambda b: (0, 0)),
            pl.BlockSpec((_D, 1), lambda b: (0, 0)),
        ],
        out_specs=pl.BlockSpec((_GB, _D, _P), lambda b: (b, 0, 0)),
        out_shape=jax.ShapeDtypeStruct((_B, _D, _P), jnp.float32),
    )(samp, w3, obt, nwt, nbt)


def kernel(x, conv_w, conv_b, pe_norm_w, pe_norm_b, off_w, out_w, out_b,
           norm_w, norm_b):
    patches = (x.astype(jnp.bfloat16)
               .reshape(_B, _C, _PC, _PP, _PC, _PP)
               .transpose(0, 2, 4, 1, 3, 5)
               .reshape(_B, _P, _K))
    cwf = conv_w.reshape(_D, _K).T.astype(jnp.bfloat16)
    idx, wgt = _run_head(patches, cwf,
                         conv_b.reshape(1, _D),
                         pe_norm_w.reshape(1, _D),
                         pe_norm_b.reshape(1, _D),
                         off_w)
    pad = _PPAD - _P
    idx_t = jnp.pad(idx.transpose(0, 2, 1), ((0, 0), (0, 0), (0, pad)))
    wgt_t = jnp.pad(wgt.transpose(0, 2, 1), ((0, 0), (0, 0), (0, pad)))
    samp = _sc_bilinear(x.reshape(_B * _C, _HW), idx_t, wgt_t)
    w3 = _run_w3(out_w)
    out_t = _run_tail(samp, w3,
                      out_b.reshape(_D, 1),
                      norm_w.reshape(_D, 1),
                      norm_b.reshape(_D, 1))
    return out_t.transpose(0, 2, 1)


# speedup vs baseline: 1.2369x; 1.0025x over previous
import functools

import jax
import jax.numpy as jnp
from jax import lax
from jax.experimental import pallas as pl
from jax.experimental.pallas import tpu as pltpu
from jax.experimental.pallas import tpu_sc as plsc

_B = 32
_C = 3
_H = 224
_W = 224
_PC = 14
_PP = 16
_D = 192
_P = _PC * _PC
_K = _C * _PP * _PP
_PPAD = 208
_HW = _H * _W
_GB = 16
_GN = _B // _GB


def _erf(z):
    s = jnp.sign(z)
    a = jnp.abs(z)
    t = 1.0 / (1.0 + 0.3275911 * a)
    poly = t * (0.254829592 + t * (-0.284496736 + t * (1.421413741
                + t * (-1.453152027 + t * 1.061405429))))
    return s * (1.0 - poly * jnp.exp(-a * a))


def _layernorm(x, w, b, eps=1e-5):
    m = jnp.mean(x, axis=-1, keepdims=True)
    xc = x - m
    v = jnp.mean(xc * xc, axis=-1, keepdims=True)
    return xc * jax.lax.rsqrt(v + eps) * w + b


def _head_body(patches_ref, cw_ref, cb_ref, pw_ref, pb_ref, offw_ref,
               idx_ref, wgt_ref):
    p = patches_ref[...].reshape(_GB * _P, _K)
    feat = jnp.dot(p, cw_ref[...], preferred_element_type=jnp.float32)
    feat = feat + cb_ref[...]
    feat = _layernorm(feat, pw_ref[...], pb_ref[...])
    g = 0.5 * feat * (1.0 + _erf(feat * 0.7071067811865476))
    off = lax.dot_general(g, offw_ref[...], (((1,), (1,)), ((), ())),
                          preferred_element_type=jnp.float32)

    rid = lax.broadcasted_iota(jnp.int32, (_GB * _P, 1), 0)
    pid = rid % _P
    gx = ((pid % _PC).astype(jnp.float32) + 0.5) / _PC
    gy = ((pid // _PC).astype(jnp.float32) + 0.5) / _PC
    w0 = _PP / float(_W)
    h0 = _PP / float(_H)
    cx = gx + off[:, 0:1] * w0
    cy = gy + off[:, 1:2] * h0
    px = cx * _W - 0.5
    py = cy * _H - 0.5
    x0f = jnp.floor(px)
    y0f = jnp.floor(py)
    wx1 = px - x0f
    wy1 = py - y0f
    x0 = x0f.astype(jnp.int32)
    y0 = y0f.astype(jnp.int32)

    idx_cols = []
    wgt_cols = []
    for dy, dx, wfac in ((0, 0, (1.0 - wx1) * (1.0 - wy1)),
                         (0, 1, wx1 * (1.0 - wy1)),
                         (1, 0, (1.0 - wx1) * wy1),
                         (1, 1, wx1 * wy1)):
        ix = x0 + dx
        iy = y0 + dy
        valid = ((ix >= 0) & (ix < _W) & (iy >= 0) & (iy < _H))
        flat = (jnp.clip(iy, 0, _H - 1) * _W + jnp.clip(ix, 0, _W - 1))
        idx_cols.append(flat)
        wgt_cols.append(wfac * valid.astype(jnp.float32))
    idx_ref[...] = jnp.concatenate(idx_cols, axis=1).reshape(_GB, _P, 4)
    wgt_ref[...] = jnp.concatenate(wgt_cols, axis=1).reshape(_GB, _P, 4)


def _run_head(patches, cwf, cb, pw, pb, offw):
    return pl.pallas_call(
        _head_body,
        grid=(_GN,),
        in_specs=[
            pl.BlockSpec((_GB, _P, _K), lambda b: (b, 0, 0)),
            pl.BlockSpec((_K, _D), lambda b: (0, 0)),
            pl.BlockSpec((1, _D), lambda b: (0, 0)),
            pl.BlockSpec((1, _D), lambda b: (0, 0)),
            pl.BlockSpec((1, _D), lambda b: (0, 0)),
            pl.BlockSpec((4, _D), lambda b: (0, 0)),
        ],
        out_specs=[
            pl.BlockSpec((_GB, _P, 4), lambda b: (b, 0, 0)),
            pl.BlockSpec((_GB, _P, 4), lambda b: (b, 0, 0)),
        ],
        out_shape=[
            jax.ShapeDtypeStruct((_B, _P, 4), jnp.int32),
            jax.ShapeDtypeStruct((_B, _P, 4), jnp.float32),
        ],
    )(patches, cwf, cb, pw, pb, offw)


def _sc_bilinear(xplanes, idx, wgt):
    mesh = plsc.VectorSubcoreMesh(core_axis_name="c", subcore_axis_name="s")

    @functools.partial(
        pl.kernel,
        mesh=mesh,
        compiler_params=pltpu.CompilerParams(needs_layout_passes=False),
        out_type=jax.ShapeDtypeStruct((_B, _C, _PPAD), jnp.float32),
        scratch_types=[
            pltpu.VMEM((_HW,), jnp.float32),
            pltpu.VMEM((4, _PPAD), jnp.int32),
            pltpu.VMEM((4, _PPAD), jnp.float32),
            pltpu.VMEM((_C, _PPAD), jnp.float32),
        ],
    )
    def k(x_hbm, idx_hbm, wgt_hbm, out_hbm, plane_v, idx_v, wgt_v, samp_v):
        b = lax.axis_index("s") * 2 + lax.axis_index("c")
        pltpu.sync_copy(idx_hbm.at[b], idx_v)
        pltpu.sync_copy(wgt_hbm.at[b], wgt_v)
        for c in range(_C):
            pltpu.sync_copy(x_hbm.at[b * _C + c], plane_v)
            for j in range(_PPAD // 16):
                sl = pl.ds(j * 16, 16)
                acc = plsc.load_gather(plane_v, [idx_v[0, sl]]) * wgt_v[0, sl]
                for t in range(1, 4):
                    acc = acc + (plsc.load_gather(plane_v, [idx_v[t, sl]])
                                 * wgt_v[t, sl])
                samp_v[c, sl] = acc
        pltpu.sync_copy(samp_v, out_hbm.at[b])

    return k(xplanes, idx, wgt)


def _w3_body(ow_ref, w3_ref):
    ow = ow_ref[...]
    kk = lax.broadcasted_iota(jnp.int32, (_K, _C), 0) % _C
    cc = lax.broadcasted_iota(jnp.int32, (_K, _C), 1)
    m = (kk == cc).astype(jnp.float32)
    w3_ref[...] = jnp.dot(ow, m, preferred_element_type=jnp.float32)


def _run_w3(ow):
    return pl.pallas_call(
        _w3_body,
        out_shape=jax.ShapeDtypeStruct((_D, _C), jnp.float32),
    )(ow)


def _tail_body(samp_ref, w3_ref, ob_ref, nw_ref, nb_ref, out_ref):
    w3 = w3_ref[...]
    obt = ob_ref[...]
    nwt = nw_ref[...]
    nbt = nb_ref[...]
    eps = 1e-5
    for i in range(_GB):
        s = samp_ref[i, :, :_P]
        pre = jnp.dot(w3, s, preferred_element_type=jnp.float32) + obt
        m = jnp.mean(pre, axis=0, keepdims=True)
        xc = pre - m
        v = jnp.mean(xc * xc, axis=0, keepdims=True)
        out_ref[i] = xc * jax.lax.rsqrt(v + eps) * nwt + nbt


def _run_tail(samp, w3, obt, nwt, nbt):
    return pl.pallas_call(
        _tail_body,
        grid=(_GN,),
        in_specs=[
            pl.BlockSpec((_GB, _C, _PPAD), lambda b: (b, 0, 0)),
            pl.BlockSpec((_D, _C), lambda b: (0, 0)),
            pl.BlockSpec((_D, 1), lambda b: (0, 0)),
            pl.BlockSpec((_D, 1), lambda b: (0, 0)),
            pl.BlockSpec((_D, 1), lambda b: (0, 0)),
        ],
        out_specs=pl.BlockSpec((_GB, _D, _P), lambda b: (b, 0, 0)),
        out_shape=jax.ShapeDtypeStruct((_B, _D, _P), jnp.float32),
    )(samp, w3, obt, nwt, nbt)


def kernel(x, conv_w, conv_b, pe_norm_w, pe_norm_b, off_w, out_w, out_b,
           norm_w, norm_b):
    patches = (x.astype(jnp.bfloat16)
               .reshape(_B, _C, _PC, _PP, _PC, _PP)
               .transpose(0, 2, 4, 1, 3, 5)
               .reshape(_B, _P, _K))
    cwf = conv_w.reshape(_D, _K).T.astype(jnp.bfloat16)
    idx, wgt = _run_head(patches, cwf,
                         conv_b.reshape(1, _D),
                         pe_norm_w.reshape(1, _D),
                         pe_norm_b.reshape(1, _D),
                         off_w)
    pad = _PPAD - _P
    idx_t = jnp.pad(idx.transpose(0, 2, 1), ((0, 0), (0, 0), (0, pad)))
    wgt_t = jnp.pad(wgt.transpose(0, 2, 1), ((0, 0), (0, 0), (0, pad)))
    samp = _sc_bilinear(x.reshape(_B * _C, _HW), idx_t, wgt_t)
    w3 = _run_w3(out_w)
    out_t = _run_tail(samp, w3,
                      out_b.reshape(_D, 1),
                      norm_w.reshape(_D, 1),
                      norm_b.reshape(_D, 1))
    return out_t.transpose(0, 2, 1)
